# baseline (speedup 1.0000x reference)
import functools
import math

import jax
import jax.numpy as jnp
from jax.experimental import pallas as pl
from jax.experimental.pallas import tpu as pltpu


def _block_kernel(x_ref, w_in_ref, b_in_ref, w_out_ref, b_out_ref,
                  gamma_ref, beta_ref, o_ref, *, nhead, eps, scale):
    L, nb, E = x_ref.shape
    hd = E // nhead

    x2d = x_ref[...].reshape(L * nb, E)

    xb = jnp.transpose(x2d.astype(jnp.float8_e4m3fn).reshape(L, nb, E),
                       (1, 0, 2)).reshape(nb * L, E)

    w_in = (w_in_ref[...] * 16.0).astype(jnp.float8_e4m3fn)
    w_out = (w_out_ref[...] * 16.0).astype(jnp.float8_e4m3fn)

    qk = jax.lax.dot_general(
        xb, w_in[:2 * E], (((1,), (1,)), ((), ())),
        preferred_element_type=jnp.float32).astype(jnp.bfloat16)
    qk = qk + (b_in_ref[:, :2 * E] * 16.0).astype(jnp.bfloat16)
    vv = jax.lax.dot_general(
        xb, w_in[2 * E:], (((1,), (1,)), ((), ())),
        preferred_element_type=jnp.float32).astype(jnp.float8_e4m3fn)

    exp2_c = scale * 1.4426950408889634 / 256.0

    heads = []
    for h in range(nhead):
        q = qk[:, h * hd:(h + 1) * hd].astype(jnp.float8_e4m3fn).reshape(nb, L, hd)
        k = qk[:, E + h * hd:E + (h + 1) * hd].astype(jnp.float8_e4m3fn).reshape(nb, L, hd)
        v = vv[:, h * hd:(h + 1) * hd].reshape(nb, L, hd)
        bv = b_in_ref[0:1, 2 * E + h * hd:2 * E + (h + 1) * hd].reshape(1, 1, hd) * 16.0

        s = jax.lax.dot_general(q, k, (((2,), (2,)), ((0,), (0,))),
                                preferred_element_type=jnp.float32)
        sb = (s * exp2_c).astype(jnp.bfloat16)
        mx = jnp.max(sb, axis=-1, keepdims=True)
        p = jnp.exp2(sb - mx)
        denom = jnp.sum(p, axis=-1, keepdims=True).astype(jnp.float32)
        o = jax.lax.dot_general(p.astype(jnp.float8_e4m3fn), v,
                                (((2,), (1,)), ((0,), (0,))),
                                preferred_element_type=jnp.float32)
        o = o * pl.reciprocal(denom, approx=True) + bv
        heads.append(o.astype(jnp.float8_e4m3fn))

    attn = jnp.concatenate(heads, axis=-1)
    attn = jnp.transpose(attn, (1, 0, 2)).reshape(L * nb, E)

    proj = jax.lax.dot_general(attn, w_out, (((1,), (1,)), ((), ())),
                               preferred_element_type=jnp.float32)
    y = x2d + (b_out_ref[...] + proj * (1.0 / 256.0))

    mu = jnp.mean(y, axis=-1, keepdims=True)
    var = jnp.mean(y * y, axis=-1, keepdims=True) - mu * mu
    yn = (y - mu) * jax.lax.rsqrt(var + eps)
    o_ref[...] = (yn * gamma_ref[...] + beta_ref[...]).reshape(L, nb, E)


def kernel(src, in_proj_weight, in_proj_bias, out_proj_weight,
           out_proj_bias, ln_weight, ln_bias, *, nhead=8, eps=1e-5,
           batch_block=32):
    L, N, E = src.shape
    hd = E // nhead
    scale = 1.0 / math.sqrt(hd)

    nb = min(batch_block, N)
    assert N % nb == 0

    b_in_row = in_proj_bias.reshape(1, 3 * E)
    b_out_row = out_proj_bias.reshape(1, E)
    gamma_row = ln_weight.reshape(1, E)
    beta_row = ln_bias.reshape(1, E)

    kern = functools.partial(_block_kernel, nhead=nhead, eps=eps, scale=scale)

    return pl.pallas_call(
        kern,
        out_shape=jax.ShapeDtypeStruct((L, N, E), jnp.float32),
        grid=(N // nb,),
        in_specs=[
            pl.BlockSpec((L, nb, E), lambda b: (0, b, 0)),
            pl.BlockSpec((3 * E, E), lambda b: (0, 0)),
            pl.BlockSpec((1, 3 * E), lambda b: (0, 0)),
            pl.BlockSpec((E, E), lambda b: (0, 0)),
            pl.BlockSpec((1, E), lambda b: (0, 0)),
            pl.BlockSpec((1, E), lambda b: (0, 0)),
            pl.BlockSpec((1, E), lambda b: (0, 0)),
        ],
        out_specs=pl.BlockSpec((L, nb, E), lambda b: (0, b, 0)),
        compiler_params=pltpu.CompilerParams(
            dimension_semantics=("arbitrary",)),
    )(src, in_proj_weight, b_in_row, out_proj_weight, b_out_row,
      gamma_row, beta_row)


# speedup vs baseline: 1.0029x; 1.0029x over previous
import functools
import math

import jax
import jax.numpy as jnp
from jax.experimental import pallas as pl
from jax.experimental.pallas import tpu as pltpu


def _block_kernel(x_ref, w_in_ref, b_in_ref, w_out_ref, b_out_ref,
                  gamma_ref, beta_ref, o_ref, *, nhead, eps, scale):
    L, nb, E = x_ref.shape
    hd = E // nhead

    x2d = x_ref[...].reshape(L * nb, E)

    xb = jnp.transpose(x2d.astype(jnp.float8_e4m3fn).reshape(L, nb, E),
                       (1, 0, 2)).reshape(nb * L, E)

    w_in = (w_in_ref[...] * 16.0).astype(jnp.float8_e4m3fn)
    w_out = (w_out_ref[...] * 16.0).astype(jnp.float8_e4m3fn)

    qkv = jax.lax.dot_general(
        xb, w_in, (((1,), (1,)), ((), ())),
        preferred_element_type=jnp.float32)
    qk = qkv[:, :2 * E].astype(jnp.bfloat16)
    qk = qk + (b_in_ref[:, :2 * E] * 16.0).astype(jnp.bfloat16)
    vv = qkv[:, 2 * E:].astype(jnp.float8_e4m3fn)

    exp2_c = scale * 1.4426950408889634 / 256.0

    heads = []
    for h in range(nhead):
        q = qk[:, h * hd:(h + 1) * hd].astype(jnp.float8_e4m3fn).reshape(nb, L, hd)
        k = qk[:, E + h * hd:E + (h + 1) * hd].astype(jnp.float8_e4m3fn).reshape(nb, L, hd)
        v = vv[:, h * hd:(h + 1) * hd].reshape(nb, L, hd)
        bv = b_in_ref[0:1, 2 * E + h * hd:2 * E + (h + 1) * hd].reshape(1, 1, hd) * 16.0

        s = jax.lax.dot_general(q, k, (((2,), (2,)), ((0,), (0,))),
                                preferred_element_type=jnp.float32)
        sb = (s * exp2_c).astype(jnp.bfloat16)
        mx = jnp.max(sb, axis=-1, keepdims=True)
        p = jnp.exp2(sb - mx)
        denom = jnp.sum(p, axis=-1, keepdims=True).astype(jnp.float32)
        o = jax.lax.dot_general(p.astype(jnp.float8_e4m3fn), v,
                                (((2,), (1,)), ((0,), (0,))),
                                preferred_element_type=jnp.float32)
        o = o * pl.reciprocal(denom, approx=True) + bv
        heads.append(o.astype(jnp.float8_e4m3fn))

    attn = jnp.concatenate(heads, axis=-1)
    attn = jnp.transpose(attn, (1, 0, 2)).reshape(L * nb, E)

    proj = jax.lax.dot_general(attn, w_out, (((1,), (1,)), ((), ())),
                               preferred_element_type=jnp.float32)
    y = x2d + (b_out_ref[...] + proj * (1.0 / 256.0))

    mu = jnp.mean(y, axis=-1, keepdims=True)
    var = jnp.mean(y * y, axis=-1, keepdims=True) - mu * mu
    yn = (y - mu) * jax.lax.rsqrt(var + eps)
    o_ref[...] = (yn * gamma_ref[...] + beta_ref[...]).reshape(L, nb, E)


def kernel(src, in_proj_weight, in_proj_bias, out_proj_weight,
           out_proj_bias, ln_weight, ln_bias, *, nhead=8, eps=1e-5,
           batch_block=32):
    L, N, E = src.shape
    hd = E // nhead
    scale = 1.0 / math.sqrt(hd)

    nb = min(batch_block, N)
    assert N % nb == 0

    b_in_row = in_proj_bias.reshape(1, 3 * E)
    b_out_row = out_proj_bias.reshape(1, E)
    gamma_row = ln_weight.reshape(1, E)
    beta_row = ln_bias.reshape(1, E)

    kern = functools.partial(_block_kernel, nhead=nhead, eps=eps, scale=scale)

    return pl.pallas_call(
        kern,
        out_shape=jax.ShapeDtypeStruct((L, N, E), jnp.float32),
        grid=(N // nb,),
        in_specs=[
            pl.BlockSpec((L, nb, E), lambda b: (0, b, 0)),
            pl.BlockSpec((3 * E, E), lambda b: (0, 0)),
            pl.BlockSpec((1, 3 * E), lambda b: (0, 0)),
            pl.BlockSpec((E, E), lambda b: (0, 0)),
            pl.BlockSpec((1, E), lambda b: (0, 0)),
            pl.BlockSpec((1, E), lambda b: (0, 0)),
            pl.BlockSpec((1, E), lambda b: (0, 0)),
        ],
        out_specs=pl.BlockSpec((L, nb, E), lambda b: (0, b, 0)),
        compiler_params=pltpu.CompilerParams(
            dimension_semantics=("arbitrary",)),
    )(src, in_proj_weight, b_in_row, out_proj_weight, b_out_row,
      gamma_row, beta_row)
